# baseline (speedup 1.0000x reference)
import jax
import jax.numpy as jnp
import numpy as np
from jax import lax
from jax.experimental import pallas as pl
from jax.experimental.pallas import tpu as pltpu
from jax.experimental.pallas import tpu_sc as plsc

B = 16
N_AGV = 16
N_STAT = 10
F = 16
GRID = 32
N = GRID * GRID
D = 64
H = 128
NB = 32
M_TOT = 26
POS_IDX = (2, 4, 6, 8, 10, 12, 14)
NW = 32
ROWS = B * N // NW
PAD = GRID
NEGF = -1e30


def _embed_kernel(o_m, o_a, o_s, w1m, w1a, w1s,
                  b1m, b1a, b1s, w2m, w2a, w2s, b2m, b2a, b2s,
                  nodes_ref, p_ref, out_ref):
    nodes_blk = nodes_ref[:]
    p = p_ref[:]

    def group(o_ref, w1_ref, b1_ref, w2_ref, b2_ref, pbase, m):
        rows = o_ref.shape[0]
        f = o_ref.shape[1]
        obs = o_ref[:]
        x = jnp.concatenate(
            [jnp.broadcast_to(obs[None], (NB, rows, f)),
             jnp.broadcast_to(nodes_blk[:, None, :], (NB, rows, 2))],
            axis=-1)
        h1 = jnp.dot(x.reshape(NB * rows, f + 2), w1_ref[:],
                     preferred_element_type=jnp.float32) + b1_ref[:]
        h1 = h1.reshape(NB, rows, H)
        mu1 = jnp.mean(h1, axis=(1, 2), keepdims=True)
        ctr = h1 - mu1
        var1 = jnp.mean(ctr * ctr, axis=(1, 2), keepdims=True)
        g1 = p[:, pbase + 0].reshape(NB, 1, 1)
        be1 = p[:, pbase + 1].reshape(NB, 1, 1)
        h1 = g1 * ctr / jnp.sqrt(var1 + 1e-5) + be1
        h1 = jnp.maximum(h1, 0.0)
        h2 = jnp.dot(h1.reshape(NB * rows, H), w2_ref[:],
                     preferred_element_type=jnp.float32) + b2_ref[:]
        h2 = h2.reshape(NB, rows, D)
        mu2 = jnp.mean(h2, axis=(1, 2), keepdims=True)
        ctr2 = h2 - mu2
        var2 = jnp.mean(ctr2 * ctr2, axis=(1, 2), keepdims=True)
        g2 = p[:, pbase + 2].reshape(NB, 1, 1)
        be2 = p[:, pbase + 3].reshape(NB, 1, 1)
        h2n = g2 * ctr2 / jnp.sqrt(var2 + 1e-5) + be2
        h2n = jnp.maximum(h2n, 0.0)
        acc = h2n[:, 0:B, :]
        for k in range(1, m):
            acc = acc + h2n[:, k * B:(k + 1) * B, :]
        return acc

    gm = group(o_m, w1m, b1m, w2m, b2m, 0, 1)
    ga = group(o_a, w1a, b1a, w2a, b2a, 4, N_AGV - 1)
    gs = group(o_s, w1s, b1s, w2s, b2s, 8, N_STAT)
    out_ref[:] = (gm + ga + gs) * (1.0 / M_TOT)


def _pos_encode(o, idxs, W_pos):
    parts = [o] + [o[..., i:i + 2] @ W_pos for i in idxs]
    return jnp.concatenate(parts, axis=-1)


def _stage_a(agvs, stat, nodes, edge_index, W_pos,
             m_W1, m_b1, m_g1, m_be1, m_W2, m_b2, m_g2, m_be2,
             a_W1, a_b1, a_g1, a_be1, a_W2, a_b2, a_g2, a_be2,
             s_W1, s_b1, s_g1, s_be1, s_W2, s_b2, s_g2, s_be2,
             *unused):
    f32 = jnp.float32
    o_m = _pos_encode(agvs[:, :1], POS_IDX, W_pos)[:, 0]
    o_a = _pos_encode(agvs[:, 1:], POS_IDX, W_pos)
    o_a = o_a.transpose(1, 0, 2).reshape((N_AGV - 1) * B, 30)
    o_s = _pos_encode(stat, (0,), W_pos)
    o_s = o_s.transpose(1, 0, 2).reshape(N_STAT * B, 18)
    p = jnp.stack([m_g1, m_be1, m_g2, m_be2,
                   a_g1, a_be1, a_g2, a_be2,
                   s_g1, s_be1, s_g2, s_be2], axis=1)

    def full(shape):
        return pl.BlockSpec(shape, lambda i: (0,) * len(shape))

    node_info = pl.pallas_call(
        _embed_kernel,
        grid=(N // NB,),
        in_specs=[
            full((B, 30)), full(((N_AGV - 1) * B, 30)), full((N_STAT * B, 18)),
            full((32, H)), full((32, H)), full((20, H)),
            full((1, H)), full((1, H)), full((1, H)),
            full((H, D)), full((H, D)), full((H, D)),
            full((1, D)), full((1, D)), full((1, D)),
            pl.BlockSpec((NB, 2), lambda i: (i, 0)),
            pl.BlockSpec((NB, 12), lambda i: (i, 0)),
        ],
        out_specs=pl.BlockSpec((NB, B, D), lambda i: (i, 0, 0)),
        out_shape=jax.ShapeDtypeStruct((N, B, D), f32),
    )(o_m, o_a, o_s, m_W1, a_W1, s_W1,
      m_b1.reshape(1, H), a_b1.reshape(1, H), s_b1.reshape(1, H),
      m_W2, a_W2, s_W2,
      m_b2.reshape(1, D), a_b2.reshape(1, D), s_b2.reshape(1, D),
      nodes, p)

    return node_info.transpose(1, 0, 2)


def _logits_kernel(x_ref, mask_ref, wl_ref, wr_ref, att_ref, comb_ref):
    masks = mask_ref[:]
    m_r = masks[:, 0:1] > 0.5
    m_l = masks[:, 1:2] > 0.5
    m_d = masks[:, 2:3] > 0.5
    m_u = masks[:, 3:4] > 0.5
    NEG = jnp.float32(NEGF)

    def shift(x, k):
        return jnp.concatenate([x[k:], x[:k]], axis=0)

    x = x_ref[0]
    xl = jnp.dot(x, wl_ref[:], preferred_element_type=jnp.float32)
    xr = jnp.dot(x, wr_ref[:], preferred_element_type=jnp.float32)

    def esum(xs):
        z = xs + xr
        z = jnp.where(z >= 0, z, 0.2 * z)
        return jnp.dot(z, att_ref[:], preferred_element_type=jnp.float32)

    e0 = esum(xl)
    er = jnp.where(m_r, esum(shift(xl, 1)), NEG)
    el = jnp.where(m_l, esum(shift(xl, N - 1)), NEG)
    ed = jnp.where(m_d, esum(shift(xl, GRID)), NEG)
    eu = jnp.where(m_u, esum(shift(xl, N - GRID)), NEG)
    emax = jnp.maximum(jnp.maximum(jnp.maximum(e0, er),
                                   jnp.maximum(el, ed)), eu)
    w0 = jnp.exp(e0 - emax)
    wr_ = jnp.exp(er - emax)
    wl_ = jnp.exp(el - emax)
    wd_ = jnp.exp(ed - emax)
    wu_ = jnp.exp(eu - emax)
    inv = 1.0 / (w0 + wr_ + wl_ + wd_ + wu_ + 1e-16)
    pad = jnp.zeros((N, 128 - D - 5), jnp.float32)
    comb_ref[0] = jnp.concatenate([xl, w0 * inv, wr_ * inv, wl_ * inv,
                                   wd_ * inv, wu_ * inv, pad], axis=1)


def _tc_logits(x, masks, Wl, Wr, att):
    blk = lambda s: pl.BlockSpec((1,) + s, lambda i: (i, 0, 0))
    full = lambda s: pl.BlockSpec(s, lambda i: (0,) * len(s))
    return pl.pallas_call(
        _logits_kernel,
        grid=(B,),
        in_specs=[blk((N, D)), full((N, 4)), full((D, D)), full((D, D)),
                  full((D, 1))],
        out_specs=blk((N, 128)),
        out_shape=jax.ShapeDtypeStruct((B, N, 128), jnp.float32),
    )(x, masks, Wl, Wr, att.reshape(D, 1))


def _sc_edge_body(comb_hbm, out_hbm, comb_v, out_v, sem):
    core = lax.axis_index("c")
    sub = lax.axis_index("s")
    wid = sub * 2 + core
    b = wid // 2
    half = wid % 2
    lo = half * ROWS
    s0 = pl.multiple_of(jnp.where(half == 0, PAD, 0), 8)
    src0 = pl.multiple_of(jnp.where(half == 0, 0, lo - PAD), 8)
    length = ROWS + PAD
    zeros16 = jnp.zeros((16,), jnp.float32)

    def zero_pad(j, carry):
        for k in range(8):
            comb_v[j, pl.ds(k * 16, 16)] = zeros16
            comb_v[ROWS + PAD + j, pl.ds(k * 16, 16)] = zeros16
        return carry

    lax.fori_loop(0, PAD, zero_pad, 0)
    pltpu.sync_copy(comb_hbm.at[b, pl.ds(src0, length)],
                    comb_v.at[pl.ds(s0, length)])

    def body(i, carry):
        base = i + PAD
        av = comb_v[base, pl.ds(D, 16)]
        a0 = av[0]
        a1 = av[1]
        a2 = av[2]
        a3 = av[3]
        a4 = av[4]
        orow = lax.div(i, 2)
        ocol = lax.rem(i, 2) * D
        for k in range(4):
            sl = pl.ds(k * 16, 16)
            out_v[orow, pl.ds(ocol + k * 16, 16)] = (
                a0 * comb_v[base, sl]
                + a1 * comb_v[base + 1, sl]
                + a2 * comb_v[base - 1, sl]
                + a3 * comb_v[base + GRID, sl]
                + a4 * comb_v[base - GRID, sl])
        return carry

    lax.fori_loop(0, ROWS, body, 0)
    pltpu.sync_copy(out_v,
                    out_hbm.at[b, pl.ds(pl.multiple_of(lo // 2, 8),
                                        ROWS // 2)])


def _sc_edge(comb):
    mesh = plsc.VectorSubcoreMesh(core_axis_name="c", subcore_axis_name="s")
    out = pl.kernel(
        _sc_edge_body,
        out_type=jax.ShapeDtypeStruct((B, N // 2, 128), jnp.float32),
        mesh=mesh,
        scratch_types=[
            pltpu.VMEM((ROWS + 2 * PAD, 128), jnp.float32),
            pltpu.VMEM((ROWS // 2, 128), jnp.float32),
            pltpu.SemaphoreType.DMA,
        ],
    )(comb)
    return out.reshape(B, N, D)


def _gnorm_kernel(m_ref, bb_ref, gn_ref, y_ref):
    o = m_ref[0] + bb_ref[:]
    gn = gn_ref[:]
    mu = jnp.mean(o, axis=0, keepdims=True)
    sub = o - gn[2] * mu
    var = jnp.mean(sub * sub, axis=0, keepdims=True)
    y_ref[0] = gn[0] * sub / jnp.sqrt(var + 1e-5) + gn[1]


def _tc_gnorm(msg, bb, gn):
    blk = pl.BlockSpec((1, N, D), lambda i: (i, 0, 0))
    return pl.pallas_call(
        _gnorm_kernel,
        grid=(B,),
        in_specs=[blk, pl.BlockSpec((1, D), lambda i: (0, 0)),
                  pl.BlockSpec((3, D), lambda i: (0, 0))],
        out_specs=blk,
        out_shape=jax.ShapeDtypeStruct((B, N, D), jnp.float32),
    )(msg, bb, gn)


def _stage_b(x, c1_Wl, c1_Wr, c1_att, c1_b, gn1_g, gn1_b, gn1_a,
             c2_Wl, c2_Wr, c2_att, c2_b, gn2_g, gn2_b, gn2_a):
    idx = np.arange(N)
    cc, rr = idx % GRID, idx // GRID
    masks = jnp.asarray(np.stack([cc < GRID - 1, cc > 0,
                                  rr < GRID - 1, rr > 0],
                                 axis=1).astype(np.float32))
    gn1 = jnp.stack([gn1_g, gn1_b, gn1_a])
    gn2 = jnp.stack([gn2_g, gn2_b, gn2_a])
    comb1 = _tc_logits(x, masks, c1_Wl, c1_Wr, c1_att)
    msg1 = _sc_edge(comb1)
    y1 = _tc_gnorm(msg1, c1_b.reshape(1, D), gn1)
    comb2 = _tc_logits(y1, masks, c2_Wl, c2_Wr, c2_att)
    msg2 = _sc_edge(comb2)
    return _tc_gnorm(msg2, c2_b.reshape(1, D), gn2)


def kernel(agvs, stat, nodes, edge_index, W_pos,
           m_W1, m_b1, m_g1, m_be1, m_W2, m_b2, m_g2, m_be2,
           a_W1, a_b1, a_g1, a_be1, a_W2, a_b2, a_g2, a_be2,
           s_W1, s_b1, s_g1, s_be1, s_W2, s_b2, s_g2, s_be2,
           c1_Wl, c1_Wr, c1_att, c1_b, gn1_g, gn1_b, gn1_a,
           c2_Wl, c2_Wr, c2_att, c2_b, gn2_g, gn2_b, gn2_a):
    node_info = _stage_a(agvs, stat, nodes, edge_index, W_pos,
                         m_W1, m_b1, m_g1, m_be1, m_W2, m_b2, m_g2, m_be2,
                         a_W1, a_b1, a_g1, a_be1, a_W2, a_b2, a_g2, a_be2,
                         s_W1, s_b1, s_g1, s_be1, s_W2, s_b2, s_g2, s_be2)
    return _stage_b(node_info,
                    c1_Wl, c1_Wr, c1_att, c1_b, gn1_g, gn1_b, gn1_a,
                    c2_Wl, c2_Wr, c2_att, c2_b, gn2_g, gn2_b, gn2_a)
